# baseline (speedup 1.0000x reference)
import functools

import jax
import jax.numpy as jnp
from jax import lax
from jax.experimental import pallas as pl
from jax.experimental.pallas import tpu as pltpu
from jax.experimental.pallas import tpu_sc as plsc

F = 26
V = 100000
D = 16
B = 16384
TOTAL = B * F
NC, NS, L = 2, 16, 16
NW = NC * NS
PER_W = TOTAL // NW
CHUNK = 1664
NCH = PER_W // CHUNK
GSZ = 128
NG = CHUNK // GSZ

_mesh = plsc.VectorSubcoreMesh(core_axis_name="c", subcore_axis_name="s")


@functools.partial(
    pl.kernel,
    mesh=_mesh,
    compiler_params=pltpu.CompilerParams(use_tc_tiling_on_sc=False),
    out_type=jax.ShapeDtypeStruct((TOTAL, D), jnp.float32),
    scratch_types=[
        pltpu.VMEM((CHUNK,), jnp.int32),
        pltpu.VMEM((NG, GSZ), jnp.int32),
        pltpu.VMEM((CHUNK, D), jnp.float32),
        pltpu.SemaphoreType.DMA,
        pltpu.SemaphoreType.DMA,
    ],
)
def _gather_kernel(xq_hbm, table_hbm, out_hbm, idx_v, oix_v, rows_v, gsem, ssem):
    wid = lax.axis_index("s") * NC + lax.axis_index("c")
    base = wid * PER_W
    iota = lax.iota(jnp.int32, L)

    def chunk_body(c, carry):
        start = base + c * CHUNK
        copy_in = pltpu.make_async_copy(
            xq_hbm.at[pl.ds(start, CHUNK)], idx_v, gsem
        )
        copy_in.start()
        copy_in.wait()

        for j in range(NG):
            def vec_body(r, carry2):
                s = pl.ds(j * GSZ + r * L, L)
                p = start + j * GSZ + r * L + iota
                f = lax.shift_right_logical(p, 14)
                b = lax.bitwise_and(p, 16383)
                idx_v[s] = idx_v[s] + f * V
                oix_v[j, pl.ds(r * L, L)] = b * F + f
                return carry2

            lax.fori_loop(0, GSZ // L, vec_body, 0)

        gathers = []
        for j in range(NG):
            s = pl.ds(j * GSZ, GSZ)
            gathers.append(
                pltpu.async_copy(table_hbm.at[idx_v.at[s]], rows_v.at[s], gsem)
            )
        for d in gathers:
            d.wait()

        scatters = []
        for j in range(NG):
            s = pl.ds(j * GSZ, GSZ)
            scatters.append(
                pltpu.async_copy(rows_v.at[s], out_hbm.at[oix_v.at[j]], ssem)
            )
        for d in scatters:
            d.wait()
        return carry

    lax.fori_loop(0, NCH, chunk_body, 0)


def kernel(x, tables):
    xq = x.astype(jnp.int32).T.reshape(TOTAL)
    flat_tables = tables.reshape(F * V, D)
    out = _gather_kernel(xq, flat_tables)
    return out.reshape(B, F * D)
